# baseline (speedup 1.0000x reference)
import jax
import jax.numpy as jnp
from jax.experimental import pallas as pl


def _h_kernel(x_ref, w_ref, b_ref, h_ref):
    h = jnp.dot(x_ref[...], w_ref[...], preferred_element_type=jnp.float32)
    h_ref[...] = (h + b_ref[...]).astype(jnp.bfloat16)


def _agg_kernel(adj_ref, h_ref, out_ref):
    a = adj_ref[...].astype(jnp.bfloat16)
    acc = jnp.dot(a, h_ref[...], preferred_element_type=jnp.float32)
    out_ref[...] = jnp.maximum(acc, 0.0)


def kernel(x, adj, W, b):
    n, f_in = x.shape
    f_out = W.shape[1]

    bm_h = 2000
    h = pl.pallas_call(
        _h_kernel,
        grid=(n // bm_h,),
        in_specs=[
            pl.BlockSpec((bm_h, f_in), lambda i: (i, 0)),
            pl.BlockSpec((f_in, f_out), lambda i: (0, 0)),
            pl.BlockSpec((1, f_out), lambda i: (0, 0)),
        ],
        out_specs=pl.BlockSpec((bm_h, f_out), lambda i: (i, 0)),
        out_shape=jax.ShapeDtypeStruct((n, f_out), jnp.bfloat16),
    )(x, W, b.reshape(1, f_out))

    bm = 400
    out = pl.pallas_call(
        _agg_kernel,
        grid=(n // bm,),
        in_specs=[
            pl.BlockSpec((bm, n), lambda i: (i, 0)),
            pl.BlockSpec((n, f_out), lambda i: (0, 0)),
        ],
        out_specs=pl.BlockSpec((bm, f_out), lambda i: (i, 0)),
        out_shape=jax.ShapeDtypeStruct((n, f_out), jnp.float32),
    )(adj, h)

    return (out, adj)
